# baseline (speedup 1.0000x reference)
import functools

import jax
import jax.numpy as jnp
from jax import lax
from jax.experimental import pallas as pl
from jax.experimental.pallas import tpu as pltpu
from jax.experimental.pallas import tpu_sc as plsc

B = 16384
D = 64
NC = 2
NS = 16
NW = NC * NS
BPW = B // NW
GCH = 128
NG = BPW // GCH
LANES = 16


_mesh = plsc.VectorSubcoreMesh(core_axis_name="c", subcore_axis_name="s")


@functools.partial(
    pl.kernel,
    mesh=_mesh,
    out_type=jax.ShapeDtypeStruct((B, D), jnp.float32),
    scratch_types=[
        pltpu.VMEM((NG, GCH), jnp.int32),
        pltpu.VMEM((BPW, D), jnp.float32),
        pltpu.SemaphoreType.DMA,
    ],
    compiler_params=pltpu.CompilerParams(use_tc_tiling_on_sc=False),
)
def _gate_sc(idx_hbm, table_hbm, out_hbm, idx_v, rows_v, sem):
    wid = lax.axis_index("s") * NC + lax.axis_index("c")
    base = wid * BPW

    pltpu.sync_copy(idx_hbm.at[pl.ds(wid * NG, NG)], idx_v)

    copies = []
    for j in range(NG):
        copies.append(
            pltpu.async_copy(
                table_hbm.at[idx_v.at[j]],
                rows_v.at[pl.ds(j * GCH, GCH)],
                sem,
            )
        )
    for c in copies:
        c.wait()

    def row_body(i, _):
        for j in range(D // LANES):
            x = rows_v[i, pl.ds(j * LANES, LANES)]
            rows_v[i, pl.ds(j * LANES, LANES)] = 1.0 / (1.0 + jnp.exp(-x))
        return 0

    lax.fori_loop(0, BPW, row_body, 0)

    pltpu.sync_copy(rows_v, out_hbm.at[pl.ds(base, BPW)])


def kernel(X, Y, gate_theta):
    del X
    idx = Y.astype(jnp.int32).reshape(NW * NG, GCH)
    return _gate_sc(idx, gate_theta)


# speedup vs baseline: 2.0356x; 2.0356x over previous
import functools

import jax
import jax.numpy as jnp
from jax import lax
from jax.experimental import pallas as pl
from jax.experimental.pallas import tpu as pltpu
from jax.experimental.pallas import tpu_sc as plsc

B = 16384
D = 64
NUM_E = 1000000
NC = 2
NS = 16
NW = NC * NS
SEG = B // NW
CHW = 256
TAIL_C = NUM_E // CHW
TAIL_START = TAIL_C * CHW
ROW_PAD = 128

_mesh = plsc.VectorSubcoreMesh(core_axis_name="c", subcore_axis_name="s")
_params = pltpu.CompilerParams(
    use_tc_tiling_on_sc=True, needs_layout_passes=False
)


def _lane(vec, lane):
    sel = lax.broadcasted_iota(jnp.int32, (16,), 0) == lane
    return jnp.sum(jnp.where(sel, vec, 0))


@functools.partial(
    pl.kernel,
    mesh=_mesh,
    out_type=jax.ShapeDtypeStruct((B, ROW_PAD), jnp.float32),
    scratch_types=[
        pltpu.VMEM((B,), jnp.int32),
        pltpu.VMEM((2, D, CHW), jnp.float32),
        pltpu.VMEM((SEG, ROW_PAD), jnp.float32),
        pltpu.SemaphoreType.DMA,
        pltpu.SemaphoreType.DMA,
        pltpu.SemaphoreType.DMA,
    ],
    compiler_params=_params,
)
def _gather_sigmoid(es_hbm, tbl_t_hbm, tail_hbm, res_hbm,
                    es_v, slab_v, out_v, sem0, sem1, sem2):
    wid = lax.axis_index("s") * NC + lax.axis_index("c")
    seg0 = wid * SEG

    pltpu.sync_copy(es_hbm, es_v)

    e_first = _lane(es_v[pl.ds(seg0, 16)], 0)
    e_last = _lane(es_v[pl.ds(seg0 + SEG - 16, 16)], 15)
    c_lo = e_first // CHW
    c_hi = e_last // CHW
    cnt = c_hi - c_lo + 1

    sems = (sem0, sem1)

    def start_chunk(k, p):
        c = c_lo + k

        @pl.when(c != TAIL_C)
        def _():
            pltpu.async_copy(
                tbl_t_hbm.at[:, pl.ds(c * CHW, CHW)], slab_v.at[p], sems[p]
            )

        @pl.when(c == TAIL_C)
        def _():
            pltpu.async_copy(
                tail_hbm, slab_v.at[p, :, pl.ds(0, 128)], sems[p]
            )

    def wait_chunk(k, p):
        c = c_lo + k

        @pl.when(c != TAIL_C)
        def _():
            pltpu.make_async_copy(
                tbl_t_hbm.at[:, pl.ds(c * CHW, CHW)], slab_v.at[p], sems[p]
            ).wait()

        @pl.when(c == TAIL_C)
        def _():
            pltpu.make_async_copy(
                tail_hbm, slab_v.at[p, :, pl.ds(0, 128)], sems[p]
            ).wait()

    def process_chunk(k, p, g):
        c = c_lo + k
        eb = c * CHW
        hi = eb + jnp.where(c == TAIL_C, NUM_E - TAIL_START, CHW)
        lanes = lax.broadcasted_iota(jnp.int32, (16,), 0)

        def cond(carry):
            _, done = carry
            return jnp.logical_not(done)

        def body(carry):
            g_, done = carry
            ev = es_v[pl.ds(g_ * 16, 16)]
            in_mask = (ev >= eb) & (ev < hi)
            el = jnp.where(in_mask, ev - eb, 0)
            rows = jnp.where(in_mask, (g_ * 16 - seg0) + lanes, 0)
            pv = jnp.full((16,), p, dtype=jnp.int32)
            for h in range(D):
                hv = jnp.full((16,), h, dtype=jnp.int32)
                v = plsc.load_gather(slab_v, [pv, hv, el])
                v = 1.0 / (1.0 + jnp.exp(-v))
                plsc.store_scatter(out_v, [rows, hv], v, mask=in_mask)
            adv = jnp.max(ev) < hi
            g_n = jnp.where(adv, g_ + 1, g_)
            done_n = jnp.logical_not(adv) | (g_n * 16 >= seg0 + SEG)
            return (g_n, done_n)

        g, _ = lax.while_loop(cond, body, (g, jnp.bool_(False)))
        return g

    start_chunk(0, 0)

    def pair_body(j, g):
        k0 = 2 * j
        k1 = 2 * j + 1

        @pl.when(k1 < cnt)
        def _():
            start_chunk(k1, 1)

        def do0(g_):
            wait_chunk(k0, 0)
            return process_chunk(k0, 0, g_)

        g = lax.cond(k0 < cnt, do0, lambda g_: g_, g)

        @pl.when(k1 + 1 < cnt)
        def _():
            start_chunk(k1 + 1, 0)

        def do1(g_):
            wait_chunk(k1, 1)
            return process_chunk(k1, 1, g_)

        g = lax.cond(k1 < cnt, do1, lambda g_: g_, g)
        return g

    lax.fori_loop(0, (cnt + 1) // 2, pair_body, seg0 // 16)

    pltpu.async_copy(out_v, res_hbm.at[pl.ds(seg0, SEG)], sem2).wait()


@functools.partial(
    pl.kernel,
    mesh=_mesh,
    out_type=jax.ShapeDtypeStruct((B, ROW_PAD), jnp.float32),
    scratch_types=[
        pltpu.VMEM((SEG,), jnp.int32),
        pltpu.VMEM((SEG, ROW_PAD), jnp.float32),
        pltpu.SemaphoreType.DMA,
    ],
    compiler_params=_params,
)
def _unsort(inv_hbm, res_hbm, out_hbm, inv_v, rows_v, sem):
    wid = lax.axis_index("s") * NC + lax.axis_index("c")
    base = wid * SEG
    pltpu.sync_copy(inv_hbm.at[pl.ds(base, SEG)], inv_v)
    copies = []
    for g in range(SEG // 16):
        iv = inv_v[pl.ds(g * 16, 16)]
        copies.append(
            pltpu.async_copy(
                res_hbm.at[iv], rows_v.at[pl.ds(g * 16, 16)], sem
            )
        )
    for c in copies:
        c.wait()
    pltpu.sync_copy(rows_v, out_hbm.at[pl.ds(base, SEG)])


def kernel(X, Y, gate_theta):
    del X
    y32 = Y.astype(jnp.int32)
    iota = lax.broadcasted_iota(jnp.int32, (B,), 0)
    es, order = lax.sort([y32, iota], num_keys=1)
    inv = jnp.zeros((B,), jnp.int32).at[order].set(iota)
    tbl_t = gate_theta.T
    tail = jnp.pad(
        tbl_t[:, TAIL_START:], ((0, 0), (0, 128 - (NUM_E - TAIL_START)))
    )
    res = _gather_sigmoid(es, tbl_t, tail)
    out = _unsort(inv, res)
    return out[:, :D]


# speedup vs baseline: 2.1084x; 1.0358x over previous
import functools

import jax
import jax.numpy as jnp
from jax import lax
from jax.experimental import pallas as pl
from jax.experimental.pallas import tpu as pltpu
from jax.experimental.pallas import tpu_sc as plsc

B = 16384
D = 64
NUM_E = 1000000
NC = 2
NS = 16
NW = NC * NS
SEG = B // NW
CHW = 384
TAIL_C = NUM_E // CHW
TAIL_START = TAIL_C * CHW
ROW_PAD = 128

_mesh = plsc.VectorSubcoreMesh(core_axis_name="c", subcore_axis_name="s")
_params = pltpu.CompilerParams(
    use_tc_tiling_on_sc=True, needs_layout_passes=False
)


def _lane(vec, lane):
    sel = lax.broadcasted_iota(jnp.int32, (16,), 0) == lane
    return jnp.sum(jnp.where(sel, vec, 0))


@functools.partial(
    pl.kernel,
    mesh=_mesh,
    out_type=jax.ShapeDtypeStruct((B, ROW_PAD), jnp.float32),
    scratch_types=[
        pltpu.VMEM((SEG,), jnp.int32),
        pltpu.VMEM((2, 8, 8, CHW), jnp.float32),
        pltpu.VMEM((SEG, ROW_PAD), jnp.float32),
        pltpu.SemaphoreType.DMA,
        pltpu.SemaphoreType.DMA,
        pltpu.SemaphoreType.DMA,
    ],
    compiler_params=_params,
)
def _gather_sigmoid(es_hbm, tbl_hbm, tail_hbm, res_hbm,
                    es_v, slab_v, out_v, sem0, sem1, sem2):
    wid = lax.axis_index("s") * NC + lax.axis_index("c")
    seg0 = wid * SEG

    pltpu.sync_copy(es_hbm.at[pl.ds(seg0, SEG)], es_v)

    e_first = _lane(es_v[pl.ds(0, 16)], 0)
    e_last = _lane(es_v[pl.ds(SEG - 16, 16)], 15)
    c_lo = e_first // CHW
    c_hi = e_last // CHW
    cnt = c_hi - c_lo + 1

    sems = (sem0, sem1)

    def chunk_copies(k, p):
        c = c_lo + k
        return [
            pltpu.make_async_copy(
                tbl_hbm.at[r, :, pl.ds(c * CHW, CHW)],
                slab_v.at[p, r],
                sems[p],
            )
            for r in range(8)
        ]

    def tail_copies(p):
        return [
            pltpu.make_async_copy(
                tail_hbm.at[r], slab_v.at[p, r, :, pl.ds(0, 128)], sems[p]
            )
            for r in range(8)
        ]

    def start_chunk(k, p):
        c = c_lo + k

        @pl.when(c != TAIL_C)
        def _():
            for cp in chunk_copies(k, p):
                cp.start()

        @pl.when(c == TAIL_C)
        def _():
            for cp in tail_copies(p):
                cp.start()

    def wait_chunk(k, p):
        c = c_lo + k

        @pl.when(c != TAIL_C)
        def _():
            for cp in chunk_copies(k, p):
                cp.wait()

        @pl.when(c == TAIL_C)
        def _():
            for cp in tail_copies(p):
                cp.wait()

    lanes = lax.broadcasted_iota(jnp.int32, (16,), 0)

    def process_chunk(k, p, g):
        c = c_lo + k
        eb = c * CHW
        hi = eb + jnp.where(c == TAIL_C, NUM_E - TAIL_START, CHW)

        def cond(carry):
            _, done = carry
            return jnp.logical_not(done)

        def body(carry):
            g_, done = carry
            ev = es_v[pl.ds(g_ * 16, 16)]
            in_mask = (ev >= eb) & (ev < hi)
            el = ev - eb
            for ln in range(16):
                @pl.when(_lane(in_mask.astype(jnp.int32), ln) > 0)
                def _():
                    e_s = jnp.full((16,), _lane(el, ln), dtype=jnp.int32)
                    pv = jnp.full((16,), p, dtype=jnp.int32)
                    row = g_ * 16 + ln
                    for j in range(D // 16):
                        fr = (lanes + 16 * j) // 8
                        fc = (lanes + 16 * j) % 8
                        v = plsc.load_gather(slab_v, [pv, fr, fc, e_s])
                        v = 1.0 / (1.0 + jnp.exp(-v))
                        out_v[row, pl.ds(16 * j, 16)] = v
            adv = jnp.max(ev) < hi
            g_n = jnp.where(adv, g_ + 1, g_)
            done_n = jnp.logical_not(adv) | (g_n >= SEG // 16)
            return (g_n, done_n)

        g, _ = lax.while_loop(cond, body, (g, jnp.bool_(False)))
        return g

    start_chunk(0, 0)

    def pair_body(j, g):
        k0 = 2 * j
        k1 = 2 * j + 1

        @pl.when(k1 < cnt)
        def _():
            start_chunk(k1, 1)

        def do0(g_):
            wait_chunk(k0, 0)
            return process_chunk(k0, 0, g_)

        g = lax.cond(k0 < cnt, do0, lambda g_: g_, g)

        @pl.when(k1 + 1 < cnt)
        def _():
            start_chunk(k1 + 1, 0)

        def do1(g_):
            wait_chunk(k1, 1)
            return process_chunk(k1, 1, g_)

        g = lax.cond(k1 < cnt, do1, lambda g_: g_, g)
        return g

    lax.fori_loop(0, (cnt + 1) // 2, pair_body, jnp.int32(0))

    pltpu.async_copy(out_v, res_hbm.at[pl.ds(seg0, SEG)], sem2).wait()


@functools.partial(
    pl.kernel,
    mesh=_mesh,
    out_type=jax.ShapeDtypeStruct((B, ROW_PAD), jnp.float32),
    scratch_types=[
        pltpu.VMEM((SEG,), jnp.int32),
        pltpu.VMEM((SEG, ROW_PAD), jnp.float32),
        pltpu.SemaphoreType.DMA,
    ],
    compiler_params=_params,
)
def _unsort(inv_hbm, res_hbm, out_hbm, inv_v, rows_v, sem):
    wid = lax.axis_index("s") * NC + lax.axis_index("c")
    base = wid * SEG
    pltpu.sync_copy(inv_hbm.at[pl.ds(base, SEG)], inv_v)
    copies = []
    for g in range(SEG // 16):
        iv = inv_v[pl.ds(g * 16, 16)]
        copies.append(
            pltpu.async_copy(
                res_hbm.at[iv], rows_v.at[pl.ds(g * 16, 16)], sem
            )
        )
    for c in copies:
        c.wait()
    pltpu.sync_copy(rows_v, out_hbm.at[pl.ds(base, SEG)])


def kernel(X, Y, gate_theta):
    del X
    y32 = Y.astype(jnp.int32)
    iota = lax.broadcasted_iota(jnp.int32, (B,), 0)
    es, order = lax.sort([y32, iota], num_keys=1)
    inv = jnp.zeros((B,), jnp.int32).at[order].set(iota)
    tbl = gate_theta.T.reshape(8, 8, NUM_E)
    tail = jnp.pad(
        tbl[:, :, TAIL_START:], ((0, 0), (0, 0), (0, 128 - (NUM_E - TAIL_START)))
    )
    res = _gather_sigmoid(es, tbl, tail)
    out = _unsort(inv, res)
    return out[:, :D]
